# baseline (speedup 1.0000x reference)
import functools

import jax
import jax.numpy as jnp
from jax import lax
from jax.experimental import pallas as pl
from jax.experimental.pallas import tpu as pltpu
import jax.experimental.pallas.tpu_sc as plsc

_NC = 2
_NS = 16
_NW = _NC * _NS
_L = 16

_N = 10000
_E = 320000
_D = 128

_CH = 40
_EP = 327680
_CHUNKS = _EP // _CH
_CPT = _CHUNKS // _NW

_NR = 10240
_SEG = _NR // _NS
_RCH = 32

_HCH = 128
_HCPT = _EP // _HCH // _NW
_HSEG = _NR // _NS

_BR = 1000


def _sc_mesh():
  return plsc.VectorSubcoreMesh(
      core_axis_name="c", subcore_axis_name="s",
      num_cores=_NC, num_subcores=_NS)


def _hist_body(dstp_hbm, out_hbm, db0, db1, ones_v, zbuf_v, acc_sh,
               is0, is1):
  cid = lax.axis_index("c")
  sid = lax.axis_index("s")
  wid = sid * _NC + cid
  base = wid * _HCPT * _HCH

  dbs = (db0, db1)
  isems = (is0, is1)

  for j in range(_HCH // _L):
    ones_v[pl.ds(j * _L, _L)] = jnp.ones((_L,), jnp.float32)
  for j in range(_HSEG // _L):
    zbuf_v[pl.ds(j * _L, _L)] = jnp.zeros((_L,), jnp.float32)
  pltpu.sync_copy(zbuf_v, acc_sh.at[pl.ds(sid * _HSEG, _HSEG)])
  plsc.subcore_barrier()

  for k in range(2):
    pltpu.async_copy(dstp_hbm.at[pl.ds(base + k * _HCH, _HCH)], dbs[k],
                     isems[k])

  def step(m, carry):
    for j in range(2):
      k = m * 2 + j
      pltpu.make_async_copy(dstp_hbm.at[pl.ds(0, _HCH)], dbs[j],
                            isems[j]).wait()
      pltpu.sync_copy(ones_v, acc_sh.at[dbs[j]], add=True)

      @pl.when(k + 2 < _HCPT)
      def _():
        pltpu.async_copy(dstp_hbm.at[pl.ds(base + (k + 2) * _HCH, _HCH)],
                         dbs[j], isems[j])
    return carry

  lax.fori_loop(0, _HCPT // 2, step, 0)
  plsc.subcore_barrier()

  pltpu.sync_copy(acc_sh.at[pl.ds(sid * _HSEG, _HSEG)],
                  out_hbm.at[cid, pl.ds(sid * _HSEG, _HSEG)])


_hist_call = functools.partial(
    pl.kernel,
    out_type=jax.ShapeDtypeStruct((_NC, _NR), jnp.float32),
    mesh=_sc_mesh(),
    scratch_types=[
        pltpu.VMEM((_HCH,), jnp.int32),
        pltpu.VMEM((_HCH,), jnp.int32),
        pltpu.VMEM((_HCH,), jnp.float32),
        pltpu.VMEM((_HSEG,), jnp.float32),
        pltpu.VMEM_SHARED((_NR,), jnp.float32),
        pltpu.SemaphoreType.DMA,
        pltpu.SemaphoreType.DMA,
    ],
)


def _agg_body(pr_hbm, ei_hbm, out_hbm,
              eb0, eb1, eb2, eb3, eb4, eb5, eb6, eb7,
              rows0, rows1, rows2, rows3, bnc_v, acc_sh,
              es0, es1, es2, es3, es4, es5, es6, es7,
              gs0, gs1, gs2, gs3, cs0, cs1, cs2, cs3):
  cid = lax.axis_index("c")
  sid = lax.axis_index("s")
  wid = sid * _NC + cid
  base = wid * _CPT

  ebs = (eb0, eb1, eb2, eb3, eb4, eb5, eb6, eb7)
  rows = (rows0, rows1, rows2, rows3)
  esems = (es0, es1, es2, es3, es4, es5, es6, es7)
  gsems = (gs0, gs1, gs2, gs3)
  csems = (cs0, cs1, cs2, cs3)

  def zrow(r, carry):
    for j in range(_D // _L):
      bnc_v[r, pl.ds(j * _L, _L)] = jnp.zeros((_L,), jnp.float32)
    return carry

  lax.fori_loop(0, _RCH, zrow, 0)
  for t in range(_SEG // _RCH):
    pltpu.sync_copy(bnc_v, acc_sh.at[pl.ds(sid * _SEG + t * _RCH, _RCH)])
  plsc.subcore_barrier()

  def fetch_idx(k, slot):
    pltpu.async_copy(ei_hbm.at[base + k], ebs[slot], esems[slot])

  def drain_rows(slot, sem):
    pltpu.make_async_copy(pr_hbm.at[pl.ds(0, _CH)], rows[slot], sem).wait()

  for k in range(3):
    fetch_idx(k, k)

  def step(m, carry):
    for j in range(8):
      k = m * 8 + j
      jr = j & 3
      pjr = (jr - 1) & 3
      pj8 = (j - 1) & 7

      pltpu.make_async_copy(ei_hbm.at[0], ebs[j], esems[j]).wait()

      def wait_sc4():
        drain_rows(jr, csems[jr])

      if j < 4:
        @pl.when(k >= 4)
        def _():
          wait_sc4()
      else:
        wait_sc4()

      pltpu.async_copy(pr_hbm.at[ebs[j].at[0]], rows[jr], gsems[jr])

      def scat():
        drain_rows(pjr, gsems[pjr])
        pltpu.async_copy(rows[pjr], acc_sh.at[ebs[pj8].at[1]], csems[pjr],
                         add=True)

      if j == 0:
        @pl.when(k > 0)
        def _():
          scat()
      else:
        scat()

      @pl.when(k + 3 < _CPT)
      def _():
        fetch_idx(k + 3, (j + 3) & 7)
    return carry

  lax.fori_loop(0, _CPT // 8, step, 0)

  drain_rows(3, gsems[3])
  pltpu.async_copy(rows[3], acc_sh.at[ebs[7].at[1]], csems[3], add=True)
  for r in range(4):
    drain_rows(r, csems[r])
  plsc.subcore_barrier()

  pltpu.sync_copy(acc_sh.at[pl.ds(sid * _SEG, _SEG)],
                  out_hbm.at[cid, pl.ds(sid * _SEG, _SEG)])


_agg_call = functools.partial(
    pl.kernel,
    out_type=jax.ShapeDtypeStruct((_NC, _NR, _D), jnp.float32),
    mesh=_sc_mesh(),
    scratch_types=(
        [pltpu.VMEM((2, _CH), jnp.int32)] * 8
        + [pltpu.VMEM((_CH, _D), jnp.float32)] * 4
        + [
            pltpu.VMEM((_RCH, _D), jnp.float32),
            pltpu.VMEM_SHARED((_NR, _D), jnp.float32),
        ]
        + [pltpu.SemaphoreType.DMA] * 16
    ),
)


def _dinv(deg_ref):
  deg = deg_ref[:, 0:1] + deg_ref[:, 1:2] + 1.0
  return lax.rsqrt(jnp.maximum(deg, 1.0))


def _scat_sum(s_ref):
  return s_ref[0, :, :] + s_ref[1, :, :]


def _pre_body(deg_ref, x_ref, w_ref, p_ref):
  g = jnp.dot(x_ref[...], w_ref[...], preferred_element_type=jnp.float32)
  p_ref[...] = g * _dinv(deg_ref)


def _mid_body(deg_ref, s_ref, p_ref, b_ref, w_ref, o_ref):
  dinv = _dinv(deg_ref)
  agg = (_scat_sum(s_ref) + p_ref[...]) * dinv
  h = jnp.maximum(agg + b_ref[...], 0.0)
  o_ref[...] = jnp.dot(h, w_ref[...], preferred_element_type=jnp.float32) * dinv


def _post_body(deg_ref, s_ref, p_ref, b_ref, w_ref, bc_ref, o_ref):
  dinv = _dinv(deg_ref)
  h = (_scat_sum(s_ref) + p_ref[...]) * dinv + b_ref[...]
  o_ref[...] = (
      jnp.dot(h, w_ref[...], preferred_element_type=jnp.float32) + bc_ref[...])


def _tc_pre(deg2, x, w1):
  grid = (_N // _BR,)
  return pl.pallas_call(
      _pre_body,
      grid=grid,
      in_specs=[
          pl.BlockSpec((_BR, _NC), lambda i: (i, 0)),
          pl.BlockSpec((_BR, _D), lambda i: (i, 0)),
          pl.BlockSpec((_D, _D), lambda i: (0, 0)),
      ],
      out_specs=pl.BlockSpec((_BR, _D), lambda i: (i, 0)),
      out_shape=jax.ShapeDtypeStruct((_N, _D), jnp.float32),
  )(deg2, x, w1)


def _tc_mid(deg2, s, p, b, w2):
  grid = (_N // _BR,)
  return pl.pallas_call(
      _mid_body,
      grid=grid,
      in_specs=[
          pl.BlockSpec((_BR, _NC), lambda i: (i, 0)),
          pl.BlockSpec((_NC, _BR, _D), lambda i: (0, i, 0)),
          pl.BlockSpec((_BR, _D), lambda i: (i, 0)),
          pl.BlockSpec((1, _D), lambda i: (0, 0)),
          pl.BlockSpec((_D, _D), lambda i: (0, 0)),
      ],
      out_specs=pl.BlockSpec((_BR, _D), lambda i: (i, 0)),
      out_shape=jax.ShapeDtypeStruct((_N, _D), jnp.float32),
  )(deg2, s, p, b, w2)


def _tc_post(deg2, s, p, b, wc, bc, out_w):
  grid = (_N // _BR,)
  return pl.pallas_call(
      _post_body,
      grid=grid,
      in_specs=[
          pl.BlockSpec((_BR, _NC), lambda i: (i, 0)),
          pl.BlockSpec((_NC, _BR, _D), lambda i: (0, i, 0)),
          pl.BlockSpec((_BR, _D), lambda i: (i, 0)),
          pl.BlockSpec((1, _D), lambda i: (0, 0)),
          pl.BlockSpec((_D, out_w), lambda i: (0, 0)),
          pl.BlockSpec((1, out_w), lambda i: (0, 0)),
      ],
      out_specs=pl.BlockSpec((_BR, out_w), lambda i: (i, 0)),
      out_shape=jax.ShapeDtypeStruct((_N, out_w), jnp.float32),
  )(deg2, s, p, b, wc, bc)


def kernel(x, edge_index, W1, b1, W2, b2, Wc, bc):
  src = edge_index[0]
  dst = edge_index[1]

  pad = _EP - _E
  pio = jnp.arange(pad, dtype=jnp.int32)
  srcp = jnp.concatenate([src, pio & 8191])
  dstp = jnp.concatenate([dst, _N + pio % (_NR - _N)])
  ei = jnp.stack([srcp.reshape(_CHUNKS, _CH), dstp.reshape(_CHUNKS, _CH)],
                 axis=1)

  deg_parts = _hist_call(_hist_body)(dstp)
  deg2 = jnp.transpose(deg_parts)

  agg = _agg_call(_agg_body)

  p1 = _tc_pre(deg2, x, W1)
  s1 = agg(p1, ei)
  p2 = _tc_mid(deg2, s1, p1, b1.reshape(1, _D), W2)
  s2 = agg(p2, ei)
  out = _tc_post(deg2, s2, p2, b2.reshape(1, _D), Wc,
                 bc.reshape(1, -1), Wc.shape[1])
  return out


# speedup vs baseline: 1.1225x; 1.1225x over previous
import functools

import jax
import jax.numpy as jnp
from jax import lax
from jax.experimental import pallas as pl
from jax.experimental.pallas import tpu as pltpu
import jax.experimental.pallas.tpu_sc as plsc

_NC = 2
_NS = 16
_NW = _NC * _NS
_L = 16

_N = 10000
_E = 320000
_D = 128

_CH = 64
_EP = 327680
_CHUNKS = _EP // _CH
_CPT = _CHUNKS // _NW

_NR = 10240
_SEG = _NR // _NS
_RCH = 32

_HCH = 128
_HCPT = _EP // _HCH // _NW
_HSEG = _NR // _NS

_BR = 1000


def _sc_mesh():
  return plsc.VectorSubcoreMesh(
      core_axis_name="c", subcore_axis_name="s",
      num_cores=_NC, num_subcores=_NS)


def _hist_body(dstp_hbm, out_hbm, db0, db1, ones_v, zbuf_v, acc_sh,
               is0, is1):
  cid = lax.axis_index("c")
  sid = lax.axis_index("s")
  wid = sid * _NC + cid
  base = wid * _HCPT * _HCH

  dbs = (db0, db1)
  isems = (is0, is1)

  for j in range(_HCH // _L):
    ones_v[pl.ds(j * _L, _L)] = jnp.ones((_L,), jnp.float32)
  for j in range(_HSEG // _L):
    zbuf_v[pl.ds(j * _L, _L)] = jnp.zeros((_L,), jnp.float32)
  pltpu.sync_copy(zbuf_v, acc_sh.at[pl.ds(sid * _HSEG, _HSEG)])
  plsc.subcore_barrier()

  for k in range(2):
    pltpu.async_copy(dstp_hbm.at[pl.ds(base + k * _HCH, _HCH)], dbs[k],
                     isems[k])

  def step(m, carry):
    for j in range(2):
      k = m * 2 + j
      pltpu.make_async_copy(dstp_hbm.at[pl.ds(0, _HCH)], dbs[j],
                            isems[j]).wait()
      pltpu.sync_copy(ones_v, acc_sh.at[dbs[j]], add=True)

      @pl.when(k + 2 < _HCPT)
      def _():
        pltpu.async_copy(dstp_hbm.at[pl.ds(base + (k + 2) * _HCH, _HCH)],
                         dbs[j], isems[j])
    return carry

  lax.fori_loop(0, _HCPT // 2, step, 0)
  plsc.subcore_barrier()

  pltpu.sync_copy(acc_sh.at[pl.ds(sid * _HSEG, _HSEG)],
                  out_hbm.at[cid, pl.ds(sid * _HSEG, _HSEG)])


_hist_call = functools.partial(
    pl.kernel,
    out_type=jax.ShapeDtypeStruct((_NC, _NR), jnp.float32),
    mesh=_sc_mesh(),
    scratch_types=[
        pltpu.VMEM((_HCH,), jnp.int32),
        pltpu.VMEM((_HCH,), jnp.int32),
        pltpu.VMEM((_HCH,), jnp.float32),
        pltpu.VMEM((_HSEG,), jnp.float32),
        pltpu.VMEM_SHARED((_NR,), jnp.float32),
        pltpu.SemaphoreType.DMA,
        pltpu.SemaphoreType.DMA,
    ],
)


def _agg_body(pr_hbm, ei_hbm, out_hbm,
              eb0, eb1, eb2, eb3, eb4, eb5, eb6, eb7,
              rows0, rows1, rows2, rows3, bnc_v, acc_sh,
              es0, es1, es2, es3, es4, es5, es6, es7,
              gs0, gs1, gs2, gs3, cs0, cs1, cs2, cs3):
  cid = lax.axis_index("c")
  sid = lax.axis_index("s")
  wid = sid * _NC + cid
  base = wid * _CPT

  ebs = (eb0, eb1, eb2, eb3, eb4, eb5, eb6, eb7)
  rows = (rows0, rows1, rows2, rows3)
  esems = (es0, es1, es2, es3, es4, es5, es6, es7)
  gsems = (gs0, gs1, gs2, gs3)
  csems = (cs0, cs1, cs2, cs3)

  def zrow(r, carry):
    for j in range(_D // _L):
      bnc_v[r, pl.ds(j * _L, _L)] = jnp.zeros((_L,), jnp.float32)
    return carry

  lax.fori_loop(0, _RCH, zrow, 0)
  for t in range(_SEG // _RCH):
    pltpu.sync_copy(bnc_v, acc_sh.at[pl.ds(sid * _SEG + t * _RCH, _RCH)])
  plsc.subcore_barrier()

  def fetch_idx(k, slot):
    pltpu.async_copy(ei_hbm.at[base + k], ebs[slot], esems[slot])

  def drain_rows(slot, sem):
    pltpu.make_async_copy(pr_hbm.at[pl.ds(0, _CH)], rows[slot], sem).wait()

  for k in range(3):
    fetch_idx(k, k)

  def step(m, carry):
    for j in range(8):
      k = m * 8 + j
      jr = j & 3
      pjr = (jr - 1) & 3
      pj8 = (j - 1) & 7

      pltpu.make_async_copy(ei_hbm.at[0], ebs[j], esems[j]).wait()

      def wait_sc4():
        drain_rows(jr, csems[jr])

      if j < 4:
        @pl.when(k >= 4)
        def _():
          wait_sc4()
      else:
        wait_sc4()

      pltpu.async_copy(pr_hbm.at[ebs[j].at[0]], rows[jr], gsems[jr])

      def scat():
        drain_rows(pjr, gsems[pjr])
        pltpu.async_copy(rows[pjr], acc_sh.at[ebs[pj8].at[1]], csems[pjr],
                         add=True)

      if j == 0:
        @pl.when(k > 0)
        def _():
          scat()
      else:
        scat()

      @pl.when(k + 3 < _CPT)
      def _():
        fetch_idx(k + 3, (j + 3) & 7)
    return carry

  lax.fori_loop(0, _CPT // 8, step, 0)

  drain_rows(3, gsems[3])
  pltpu.async_copy(rows[3], acc_sh.at[ebs[7].at[1]], csems[3], add=True)
  for r in range(4):
    drain_rows(r, csems[r])
  plsc.subcore_barrier()

  pltpu.sync_copy(acc_sh.at[pl.ds(sid * _SEG, _SEG)],
                  out_hbm.at[cid, pl.ds(sid * _SEG, _SEG)])


_agg_call = functools.partial(
    pl.kernel,
    out_type=jax.ShapeDtypeStruct((_NC, _NR, _D), jnp.float32),
    mesh=_sc_mesh(),
    scratch_types=(
        [pltpu.VMEM((2, _CH), jnp.int32)] * 8
        + [pltpu.VMEM((_CH, _D), jnp.float32)] * 4
        + [
            pltpu.VMEM((_RCH, _D), jnp.float32),
            pltpu.VMEM_SHARED((_NR, _D), jnp.float32),
        ]
        + [pltpu.SemaphoreType.DMA] * 16
    ),
)


def _dinv(deg_ref):
  deg = deg_ref[:, 0:1] + deg_ref[:, 1:2] + 1.0
  return lax.rsqrt(jnp.maximum(deg, 1.0))


def _scat_sum(s_ref):
  return s_ref[0, :, :] + s_ref[1, :, :]


def _pre_body(deg_ref, x_ref, w_ref, p_ref):
  g = jnp.dot(x_ref[...], w_ref[...], preferred_element_type=jnp.float32)
  p_ref[...] = g * _dinv(deg_ref)


def _mid_body(deg_ref, s_ref, p_ref, b_ref, w_ref, o_ref):
  dinv = _dinv(deg_ref)
  agg = (_scat_sum(s_ref) + p_ref[...]) * dinv
  h = jnp.maximum(agg + b_ref[...], 0.0)
  o_ref[...] = jnp.dot(h, w_ref[...], preferred_element_type=jnp.float32) * dinv


def _post_body(deg_ref, s_ref, p_ref, b_ref, w_ref, bc_ref, o_ref):
  dinv = _dinv(deg_ref)
  h = (_scat_sum(s_ref) + p_ref[...]) * dinv + b_ref[...]
  o_ref[...] = (
      jnp.dot(h, w_ref[...], preferred_element_type=jnp.float32) + bc_ref[...])


def _tc_pre(deg2, x, w1):
  grid = (_N // _BR,)
  return pl.pallas_call(
      _pre_body,
      grid=grid,
      in_specs=[
          pl.BlockSpec((_BR, _NC), lambda i: (i, 0)),
          pl.BlockSpec((_BR, _D), lambda i: (i, 0)),
          pl.BlockSpec((_D, _D), lambda i: (0, 0)),
      ],
      out_specs=pl.BlockSpec((_BR, _D), lambda i: (i, 0)),
      out_shape=jax.ShapeDtypeStruct((_N, _D), jnp.float32),
  )(deg2, x, w1)


def _tc_mid(deg2, s, p, b, w2):
  grid = (_N // _BR,)
  return pl.pallas_call(
      _mid_body,
      grid=grid,
      in_specs=[
          pl.BlockSpec((_BR, _NC), lambda i: (i, 0)),
          pl.BlockSpec((_NC, _BR, _D), lambda i: (0, i, 0)),
          pl.BlockSpec((_BR, _D), lambda i: (i, 0)),
          pl.BlockSpec((1, _D), lambda i: (0, 0)),
          pl.BlockSpec((_D, _D), lambda i: (0, 0)),
      ],
      out_specs=pl.BlockSpec((_BR, _D), lambda i: (i, 0)),
      out_shape=jax.ShapeDtypeStruct((_N, _D), jnp.float32),
  )(deg2, s, p, b, w2)


def _tc_post(deg2, s, p, b, wc, bc, out_w):
  grid = (_N // _BR,)
  return pl.pallas_call(
      _post_body,
      grid=grid,
      in_specs=[
          pl.BlockSpec((_BR, _NC), lambda i: (i, 0)),
          pl.BlockSpec((_NC, _BR, _D), lambda i: (0, i, 0)),
          pl.BlockSpec((_BR, _D), lambda i: (i, 0)),
          pl.BlockSpec((1, _D), lambda i: (0, 0)),
          pl.BlockSpec((_D, out_w), lambda i: (0, 0)),
          pl.BlockSpec((1, out_w), lambda i: (0, 0)),
      ],
      out_specs=pl.BlockSpec((_BR, out_w), lambda i: (i, 0)),
      out_shape=jax.ShapeDtypeStruct((_N, out_w), jnp.float32),
  )(deg2, s, p, b, wc, bc)


def kernel(x, edge_index, W1, b1, W2, b2, Wc, bc):
  src = edge_index[0]
  dst = edge_index[1]

  pad = _EP - _E
  pio = jnp.arange(pad, dtype=jnp.int32)
  srcp = jnp.concatenate([src, pio & 8191])
  dstp = jnp.concatenate([dst, _N + pio % (_NR - _N)])
  ei = jnp.stack([srcp.reshape(_CHUNKS, _CH), dstp.reshape(_CHUNKS, _CH)],
                 axis=1)

  deg_parts = _hist_call(_hist_body)(dstp)
  deg2 = jnp.transpose(deg_parts)

  agg = _agg_call(_agg_body)

  p1 = _tc_pre(deg2, x, W1)
  s1 = agg(p1, ei)
  p2 = _tc_mid(deg2, s1, p1, b1.reshape(1, _D), W2)
  s2 = agg(p2, ei)
  out = _tc_post(deg2, s2, p2, b2.reshape(1, _D), Wc,
                 bc.reshape(1, -1), Wc.shape[1])
  return out


# speedup vs baseline: 1.2252x; 1.0915x over previous
import functools

import jax
import jax.numpy as jnp
from jax import lax
from jax.experimental import pallas as pl
from jax.experimental.pallas import tpu as pltpu
import jax.experimental.pallas.tpu_sc as plsc

_NC = 2
_NS = 16
_NW = _NC * _NS
_L = 16

_N = 10000
_E = 320000
_D = 128

_CH = 128
_EP = 327680
_CHUNKS = _EP // _CH
_CPT = _CHUNKS // _NW

_NR = 10240
_SEG = _NR // _NS
_RCH = 32

_HCH = 128
_HCPT = _EP // _HCH // _NW
_HSEG = _NR // _NS

_BR = 1000


def _sc_mesh():
  return plsc.VectorSubcoreMesh(
      core_axis_name="c", subcore_axis_name="s",
      num_cores=_NC, num_subcores=_NS)


def _hist_body(dstp_hbm, out_hbm, db0, db1, ones_v, zbuf_v, acc_sh,
               is0, is1):
  cid = lax.axis_index("c")
  sid = lax.axis_index("s")
  wid = sid * _NC + cid
  base = wid * _HCPT * _HCH

  dbs = (db0, db1)
  isems = (is0, is1)

  for j in range(_HCH // _L):
    ones_v[pl.ds(j * _L, _L)] = jnp.ones((_L,), jnp.float32)
  for j in range(_HSEG // _L):
    zbuf_v[pl.ds(j * _L, _L)] = jnp.zeros((_L,), jnp.float32)
  pltpu.sync_copy(zbuf_v, acc_sh.at[pl.ds(sid * _HSEG, _HSEG)])
  plsc.subcore_barrier()

  for k in range(2):
    pltpu.async_copy(dstp_hbm.at[pl.ds(base + k * _HCH, _HCH)], dbs[k],
                     isems[k])

  def step(m, carry):
    for j in range(2):
      k = m * 2 + j
      pltpu.make_async_copy(dstp_hbm.at[pl.ds(0, _HCH)], dbs[j],
                            isems[j]).wait()
      pltpu.sync_copy(ones_v, acc_sh.at[dbs[j]], add=True)

      @pl.when(k + 2 < _HCPT)
      def _():
        pltpu.async_copy(dstp_hbm.at[pl.ds(base + (k + 2) * _HCH, _HCH)],
                         dbs[j], isems[j])
    return carry

  lax.fori_loop(0, _HCPT // 2, step, 0)
  plsc.subcore_barrier()

  pltpu.sync_copy(acc_sh.at[pl.ds(sid * _HSEG, _HSEG)],
                  out_hbm.at[cid, pl.ds(sid * _HSEG, _HSEG)])


_hist_call = functools.partial(
    pl.kernel,
    out_type=jax.ShapeDtypeStruct((_NC, _NR), jnp.float32),
    mesh=_sc_mesh(),
    scratch_types=[
        pltpu.VMEM((_HCH,), jnp.int32),
        pltpu.VMEM((_HCH,), jnp.int32),
        pltpu.VMEM((_HCH,), jnp.float32),
        pltpu.VMEM((_HSEG,), jnp.float32),
        pltpu.VMEM_SHARED((_NR,), jnp.float32),
        pltpu.SemaphoreType.DMA,
        pltpu.SemaphoreType.DMA,
    ],
)


def _agg_body(pr_hbm, ei_hbm, out_hbm,
              eb0, eb1, eb2, eb3, rows0, rows1, bnc_v, acc_sh,
              es0, es1, es2, es3, gs0, gs1):
  cid = lax.axis_index("c")
  sid = lax.axis_index("s")
  wid = sid * _NC + cid
  base = wid * _CPT

  ebs = (eb0, eb1, eb2, eb3)
  rows = (rows0, rows1)
  esems = (es0, es1, es2, es3)
  gsems = (gs0, gs1)

  def zrow(r, carry):
    for j in range(_D // _L):
      bnc_v[r, pl.ds(j * _L, _L)] = jnp.zeros((_L,), jnp.float32)
    return carry

  lax.fori_loop(0, _RCH, zrow, 0)
  for t in range(_SEG // _RCH):
    pltpu.sync_copy(bnc_v, acc_sh.at[pl.ds(sid * _SEG + t * _RCH, _RCH)])
  plsc.subcore_barrier()

  def fetch_idx(k, slot):
    pltpu.async_copy(ei_hbm.at[base + k], ebs[slot], esems[slot])

  for k in range(3):
    fetch_idx(k, k)

  def step(m, carry):
    for j in range(4):
      k = m * 4 + j
      b = j & 1
      pltpu.make_async_copy(ei_hbm.at[0], ebs[j], esems[j]).wait()
      pltpu.async_copy(pr_hbm.at[ebs[j].at[0]], rows[b], gsems[b])

      pj = (j - 1) & 3

      def scat():
        pltpu.make_async_copy(pr_hbm.at[ebs[pj].at[0]], rows[1 - b],
                              gsems[1 - b]).wait()
        pltpu.sync_copy(rows[1 - b], acc_sh.at[ebs[pj].at[1]], add=True)

      if j == 0:
        @pl.when(k > 0)
        def _():
          scat()
      else:
        scat()

      @pl.when(k + 3 < _CPT)
      def _():
        fetch_idx(k + 3, pj)
    return carry

  lax.fori_loop(0, _CPT // 4, step, 0)

  pltpu.make_async_copy(pr_hbm.at[ebs[3].at[0]], rows[1], gsems[1]).wait()
  pltpu.sync_copy(rows[1], acc_sh.at[ebs[3].at[1]], add=True)
  plsc.subcore_barrier()

  pltpu.sync_copy(acc_sh.at[pl.ds(sid * _SEG, _SEG)],
                  out_hbm.at[cid, pl.ds(sid * _SEG, _SEG)])


_agg_call = functools.partial(
    pl.kernel,
    out_type=jax.ShapeDtypeStruct((_NC, _NR, _D), jnp.float32),
    mesh=_sc_mesh(),
    scratch_types=[
        pltpu.VMEM((2, _CH), jnp.int32),
        pltpu.VMEM((2, _CH), jnp.int32),
        pltpu.VMEM((2, _CH), jnp.int32),
        pltpu.VMEM((2, _CH), jnp.int32),
        pltpu.VMEM((_CH, _D), jnp.float32),
        pltpu.VMEM((_CH, _D), jnp.float32),
        pltpu.VMEM((_RCH, _D), jnp.float32),
        pltpu.VMEM_SHARED((_NR, _D), jnp.float32),
        pltpu.SemaphoreType.DMA,
        pltpu.SemaphoreType.DMA,
        pltpu.SemaphoreType.DMA,
        pltpu.SemaphoreType.DMA,
        pltpu.SemaphoreType.DMA,
        pltpu.SemaphoreType.DMA,
    ],
)


def _dinv(deg_ref):
  deg = deg_ref[:, 0:1] + deg_ref[:, 1:2] + 1.0
  return lax.rsqrt(jnp.maximum(deg, 1.0))


def _scat_sum(s_ref):
  return s_ref[0, :, :] + s_ref[1, :, :]


def _pre_body(deg_ref, x_ref, w_ref, p_ref):
  g = jnp.dot(x_ref[...], w_ref[...], preferred_element_type=jnp.float32)
  p_ref[...] = g * _dinv(deg_ref)


def _mid_body(deg_ref, s_ref, p_ref, b_ref, w_ref, o_ref):
  dinv = _dinv(deg_ref)
  agg = (_scat_sum(s_ref) + p_ref[...]) * dinv
  h = jnp.maximum(agg + b_ref[...], 0.0)
  o_ref[...] = jnp.dot(h, w_ref[...], preferred_element_type=jnp.float32) * dinv


def _post_body(deg_ref, s_ref, p_ref, b_ref, w_ref, bc_ref, o_ref):
  dinv = _dinv(deg_ref)
  h = (_scat_sum(s_ref) + p_ref[...]) * dinv + b_ref[...]
  o_ref[...] = (
      jnp.dot(h, w_ref[...], preferred_element_type=jnp.float32) + bc_ref[...])


def _tc_pre(deg2, x, w1):
  grid = (_N // _BR,)
  return pl.pallas_call(
      _pre_body,
      grid=grid,
      in_specs=[
          pl.BlockSpec((_BR, _NC), lambda i: (i, 0)),
          pl.BlockSpec((_BR, _D), lambda i: (i, 0)),
          pl.BlockSpec((_D, _D), lambda i: (0, 0)),
      ],
      out_specs=pl.BlockSpec((_BR, _D), lambda i: (i, 0)),
      out_shape=jax.ShapeDtypeStruct((_N, _D), jnp.float32),
  )(deg2, x, w1)


def _tc_mid(deg2, s, p, b, w2):
  grid = (_N // _BR,)
  return pl.pallas_call(
      _mid_body,
      grid=grid,
      in_specs=[
          pl.BlockSpec((_BR, _NC), lambda i: (i, 0)),
          pl.BlockSpec((_NC, _BR, _D), lambda i: (0, i, 0)),
          pl.BlockSpec((_BR, _D), lambda i: (i, 0)),
          pl.BlockSpec((1, _D), lambda i: (0, 0)),
          pl.BlockSpec((_D, _D), lambda i: (0, 0)),
      ],
      out_specs=pl.BlockSpec((_BR, _D), lambda i: (i, 0)),
      out_shape=jax.ShapeDtypeStruct((_N, _D), jnp.float32),
  )(deg2, s, p, b, w2)


def _tc_post(deg2, s, p, b, wc, bc, out_w):
  grid = (_N // _BR,)
  return pl.pallas_call(
      _post_body,
      grid=grid,
      in_specs=[
          pl.BlockSpec((_BR, _NC), lambda i: (i, 0)),
          pl.BlockSpec((_NC, _BR, _D), lambda i: (0, i, 0)),
          pl.BlockSpec((_BR, _D), lambda i: (i, 0)),
          pl.BlockSpec((1, _D), lambda i: (0, 0)),
          pl.BlockSpec((_D, out_w), lambda i: (0, 0)),
          pl.BlockSpec((1, out_w), lambda i: (0, 0)),
      ],
      out_specs=pl.BlockSpec((_BR, out_w), lambda i: (i, 0)),
      out_shape=jax.ShapeDtypeStruct((_N, out_w), jnp.float32),
  )(deg2, s, p, b, wc, bc)


def kernel(x, edge_index, W1, b1, W2, b2, Wc, bc):
  src = edge_index[0]
  dst = edge_index[1]

  pad = _EP - _E
  pio = jnp.arange(pad, dtype=jnp.int32)
  srcp = jnp.concatenate([src, pio & 8191])
  dstp = jnp.concatenate([dst, _N + pio % (_NR - _N)])
  ei = jnp.stack([srcp.reshape(_CHUNKS, _CH), dstp.reshape(_CHUNKS, _CH)],
                 axis=1)

  deg_parts = _hist_call(_hist_body)(dstp)
  deg2 = jnp.transpose(deg_parts)

  agg = _agg_call(_agg_body)

  p1 = _tc_pre(deg2, x, W1)
  s1 = agg(p1, ei)
  p2 = _tc_mid(deg2, s1, p1, b1.reshape(1, _D), W2)
  s2 = agg(p2, ei)
  out = _tc_post(deg2, s2, p2, b2.reshape(1, _D), Wc,
                 bc.reshape(1, -1), Wc.shape[1])
  return out
